# baseline (speedup 1.0000x reference)
import functools
import jax
import jax.numpy as jnp
from jax import lax
from jax.experimental import pallas as pl

TB = 1024


def _silu(t):
    return t * jax.nn.sigmoid(t)


def _moe_block(xf_ref, sw1_ref, sw3_ref, sw2_ref, rw1_ref, rw3_ref, rw2_ref,
               gw_ref, bias_ref, out_ref, idx_ref, *, n_shared, n_routed):
    xf = xf_ref[...]
    scores = jax.nn.sigmoid(
        jnp.dot(xf, gw_ref[...], preferred_element_type=jnp.float32))
    sel = scores + bias_ref[...]
    e_iota = lax.broadcasted_iota(jnp.int32, sel.shape, 1)

    v0 = jnp.max(sel, axis=1, keepdims=True)
    idx0 = jnp.min(jnp.where(sel == v0, e_iota, n_routed), axis=1)
    sel2 = jnp.where(e_iota == idx0[:, None], -jnp.inf, sel)
    v1 = jnp.max(sel2, axis=1, keepdims=True)
    idx1 = jnp.min(jnp.where(sel2 == v1, e_iota, n_routed), axis=1)

    s0 = jnp.sum(jnp.where(e_iota == idx0[:, None], scores, 0.0), axis=1)
    s1 = jnp.sum(jnp.where(e_iota == idx1[:, None], scores, 0.0), axis=1)
    denom = s0 + s1
    w0 = s0 / denom
    w1 = s1 / denom

    idx_ref[:, 0] = idx0
    idx_ref[:, 1] = idx1

    acc = jnp.zeros_like(out_ref)
    for e in range(n_shared):
        h = _silu(jnp.dot(xf, sw1_ref[e],
                          preferred_element_type=jnp.float32))
        h = h * jnp.dot(xf, sw3_ref[e],
                        preferred_element_type=jnp.float32)
        acc = acc + jnp.dot(h, sw2_ref[e],
                            preferred_element_type=jnp.float32)

    for e in range(n_routed):
        coef = w0 * (idx0 == e) + w1 * (idx1 == e)
        he = _silu(jnp.dot(xf, rw1_ref[e],
                           preferred_element_type=jnp.float32))
        he = he * jnp.dot(xf, rw3_ref[e],
                          preferred_element_type=jnp.float32)
        acc = acc + jnp.dot(coef[:, None] * he, rw2_ref[e],
                            preferred_element_type=jnp.float32)

    out_ref[...] = acc


def kernel(x, shared_w1, shared_w3, shared_w2, routed_w1, routed_w3, routed_w2,
           gate_w, expert_bias):
    b, s, d = x.shape
    t = b * s
    n_shared = shared_w1.shape[0]
    n_routed = routed_w1.shape[0]
    xf = x.reshape(t, d)

    const = lambda i: (0, 0)
    const3 = lambda i: (0, 0, 0)

    out, idx = pl.pallas_call(
        functools.partial(_moe_block, n_shared=n_shared, n_routed=n_routed),
        grid=(t // TB,),
        in_specs=[
            pl.BlockSpec((TB, d), lambda i: (i, 0)),
            pl.BlockSpec(shared_w1.shape, const3),
            pl.BlockSpec(shared_w3.shape, const3),
            pl.BlockSpec(shared_w2.shape, const3),
            pl.BlockSpec(routed_w1.shape, const3),
            pl.BlockSpec(routed_w3.shape, const3),
            pl.BlockSpec(routed_w2.shape, const3),
            pl.BlockSpec(gate_w.shape, const),
            pl.BlockSpec(expert_bias.shape, lambda i: (0,)),
        ],
        out_specs=[
            pl.BlockSpec((TB, d), lambda i: (i, 0)),
            pl.BlockSpec((TB, 2), lambda i: (i, 0)),
        ],
        out_shape=[
            jax.ShapeDtypeStruct((t, d), jnp.float32),
            jax.ShapeDtypeStruct((t, 2), jnp.int32),
        ],
    )(xf, shared_w1, shared_w3, shared_w2, routed_w1, routed_w3, routed_w2,
      gate_w, expert_bias)

    return out.reshape(b, s, d), idx.reshape(b, s, 2)
